# baseline (speedup 1.0000x reference)
import functools

import jax
import jax.numpy as jnp
from jax import lax
from jax.experimental import pallas as pl
from jax.experimental.pallas import tpu as pltpu
from jax.experimental.pallas import tpu_sc as plsc

_ROWS = 8
_N = 32768
_CHUNK = 128
_NCHUNKS = _N // _CHUNK
_GROUPS = 4
_NEG = float("-inf")


def _insert(acc, v):
    t1, t2, t3, t4, t5 = acc
    m1 = jnp.maximum(t1, v)
    r1 = jnp.minimum(t1, v)
    m2 = jnp.maximum(t2, r1)
    r2 = jnp.minimum(t2, r1)
    m3 = jnp.maximum(t3, r2)
    r3 = jnp.minimum(t3, r2)
    m4 = jnp.maximum(t4, r3)
    r4 = jnp.minimum(t4, r3)
    m5 = jnp.maximum(t5, r4)
    return (m1, m2, m3, m4, m5)


def _fold(a, b):
    return tuple(jnp.maximum(a[i], b[4 - i]) for i in range(5))


def _chunk(ref, k):
    return ref[:, k * _CHUNK:(k + 1) * _CHUNK]


def _top5_round_body(x_ref, o_ref, scratch_ref):
    iota = jax.lax.broadcasted_iota(jnp.int32, (_ROWS, _CHUNK), 1)

    per_group = _NCHUNKS // _GROUPS
    accs = [tuple(jnp.full((_ROWS, _CHUNK), _NEG) for _ in range(5))
            for _ in range(_GROUPS)]
    for j in range(per_group):
        for g in range(_GROUPS):
            accs[g] = _insert(accs[g], _chunk(x_ref, g * per_group + j))

    folded = [_fold(accs[2 * i], accs[2 * i + 1]) for i in range(_GROUPS // 2)]

    cand = jnp.concatenate([t for f in folded for t in f], axis=1)
    ccol = jax.lax.broadcasted_iota(jnp.int32, cand.shape, 1)
    vals = []
    for _ in range(5):
        m = jnp.max(cand, axis=1, keepdims=True)
        vals.append(m)
        first_col = jnp.min(
            jnp.where(cand == m, ccol, cand.shape[1]), axis=1, keepdims=True)
        cand = jnp.where(ccol == first_col, _NEG, cand)
    v4, v5 = vals[3], vals[4]

    distinct = jnp.all(v4 > v5)

    eq_cnt = jnp.zeros((_ROWS, _CHUNK), jnp.float32)
    for g in range(_GROUPS):
        for t in accs[g]:
            eq_cnt = eq_cnt + jnp.where(t == v5, 1.0, 0.0)
    unique = jnp.sum(eq_cnt, axis=1, keepdims=True) == 1.0
    rank5_hit = jnp.zeros((_ROWS, _CHUNK), jnp.bool_)
    for g in range(_GROUPS):
        rank5_hit = rank5_hit | (accs[g][4] == v5)
    no_full_group = jnp.logical_not(jnp.any(rank5_hit, axis=1, keepdims=True))

    fast = distinct & jnp.all(unique & no_full_group)
    medium = distinct & jnp.logical_not(fast)

    @pl.when(fast)
    def _():
        for k in range(_NCHUNKS):
            v = _chunk(x_ref, k)
            o_ref[:, k * _CHUNK:(k + 1) * _CHUNK] = jnp.where(
                v >= v5, jnp.round(v), 0.0)

    @pl.when(medium)
    def _():
        parts = []
        for g in range(_GROUPS):
            m = jnp.full((_ROWS, _CHUNK), _N, jnp.int32)
            for j in range(per_group):
                k = g * per_group + j
                v = _chunk(x_ref, k)
                m = jnp.minimum(m, jnp.where(v == v5, iota + k * _CHUNK, _N))
            parts.append(m)
        first_eq = jnp.min(jnp.concatenate(parts, axis=1), axis=1,
                           keepdims=True)
        for k in range(_NCHUNKS):
            v = _chunk(x_ref, k)
            keep = (v > v5) | (iota + k * _CHUNK == first_eq)
            o_ref[:, k * _CHUNK:(k + 1) * _CHUNK] = jnp.where(
                keep, jnp.round(v), 0.0)

    @pl.when(jnp.logical_not(distinct))
    def _():
        for k in range(_NCHUNKS):
            scratch_ref[:, k * _CHUNK:(k + 1) * _CHUNK] = _chunk(x_ref, k)
        for _ in range(5):
            m = jnp.full((_ROWS, _CHUNK), _NEG)
            for k in range(_NCHUNKS):
                m = jnp.maximum(m, _chunk(scratch_ref, k))
            m = jnp.max(m, axis=1, keepdims=True)
            fc = jnp.full((_ROWS, _CHUNK), _N, jnp.int32)
            for k in range(_NCHUNKS):
                fc = jnp.minimum(fc, jnp.where(
                    _chunk(scratch_ref, k) == m, iota + k * _CHUNK, _N))
            fc = jnp.min(fc, axis=1, keepdims=True)
            for k in range(_NCHUNKS):
                w = _chunk(scratch_ref, k)
                scratch_ref[:, k * _CHUNK:(k + 1) * _CHUNK] = jnp.where(
                    iota + k * _CHUNK == fc, _NEG, w)
        for k in range(_NCHUNKS):
            v = _chunk(x_ref, k)
            sel = _chunk(scratch_ref, k) == _NEG
            o_ref[:, k * _CHUNK:(k + 1) * _CHUNK] = jnp.where(
                sel, jnp.round(v), 0.0)


def _tc_top5_round(x):
    rows, n = x.shape
    grid = (rows // _ROWS,)
    return pl.pallas_call(
        _top5_round_body,
        grid=grid,
        in_specs=[pl.BlockSpec((_ROWS, n), lambda i: (i, 0))],
        out_specs=pl.BlockSpec((_ROWS, n), lambda i: (i, 0)),
        out_shape=jax.ShapeDtypeStruct(x.shape, x.dtype),
        scratch_shapes=[pltpu.VMEM((_ROWS, _N), jnp.float32)],
    )(x)


_L = 16
_NV = _N // _L
_UNROLL = 32
_MAGIC = 12582912.0
_BIG = 4194304.0
_NC = 2
_NS = 16
_NW = _NC * _NS


def _rne(v):
    y = (v + _MAGIC) - _MAGIC
    return jnp.where(jnp.abs(v) >= _BIG, v, y)


def _red(vec, op):
    vals = [vec[j] for j in range(_L)]
    while len(vals) > 1:
        vals = [op(vals[i], vals[i + 1]) for i in range(0, len(vals), 2)]
    return vals[0]


def _sc_body(x_hbm, o_hbm, buf, obuf, sem, *, rows_per_worker):
    wid = lax.axis_index("s") * _NC + lax.axis_index("c")
    io = lax.iota(jnp.int32, _L)

    def do_row(r, _):
        row = wid * rows_per_worker + r
        pltpu.sync_copy(x_hbm.at[row], buf)

        def p1(i, acc):
            for j in range(_UNROLL):
                acc = _insert(acc, buf[pl.ds((i * _UNROLL + j) * _L, _L)])
            return acc
        acc0 = tuple(jnp.full((_L,), _NEG, jnp.float32) for _ in range(5))
        t1, t2, t3, t4, t5 = lax.fori_loop(0, _NV // _UNROLL, p1, acc0)

        cands = (t1, t2, t3, t4, t5)
        work = list(cands)
        thr = jnp.float32(0)
        for _ in range(5):
            m01 = jnp.maximum(work[0], work[1])
            m23 = jnp.maximum(work[2], work[3])
            mv = jnp.maximum(jnp.maximum(m01, m23), work[4])
            thr = _red(mv, jnp.maximum)
            pos = jnp.full((_L,), 999, jnp.int32)
            for i in range(5):
                pos = jnp.minimum(
                    pos, jnp.where(work[i] == thr, io + i * _L, 999))
            g = _red(pos, jnp.minimum)
            for i in range(5):
                work[i] = jnp.where(io + i * _L == g, _NEG, work[i])

        cgt = jnp.zeros((_L,), jnp.int32)
        ceq = jnp.zeros((_L,), jnp.int32)
        for t in cands:
            cgt = cgt + jnp.where(t > thr, 1, 0)
            ceq = ceq + jnp.where(t == thr, 1, 0)
        full5 = jnp.where(t5 == thr, 1, 0)
        count_gt = _red(cgt, jnp.add)
        eq_cand = _red(ceq, jnp.add)
        full_lanes = _red(full5, jnp.add)
        r_keep = 5 - count_gt
        unique = (eq_cand == 1) & (full_lanes == 0) & (r_keep == 1)

        rthr = _rne(thr)

        def pass2_fast(_):
            def p2(i, c):
                for j in range(_UNROLL):
                    off = (i * _UNROLL + j) * _L
                    v = buf[pl.ds(off, _L)]
                    obuf[pl.ds(off, _L)] = jnp.where(v >= thr, _rne(v), 0.0)
                return c
            lax.fori_loop(0, _NV // _UNROLL, p2, jnp.int32(0))
            return 0

        def pass2_exact(_):
            def p2(i, c):
                for j in range(_UNROLL):
                    off = (i * _UNROLL + j) * _L
                    v = buf[pl.ds(off, _L)]
                    obuf[pl.ds(off, _L)] = jnp.where(v > thr, _rne(v), 0.0)
                return c
            lax.fori_loop(0, _NV // _UNROLL, p2, jnp.int32(0))

            def patch(k, prev):
                def scan_chunk(i, best):
                    for j in range(_UNROLL):
                        off = (i * _UNROLL + j) * _L
                        v = buf[pl.ds(off, _L)]
                        p = io + off
                        hit = (v == thr) & (p > prev)
                        best = jnp.minimum(best, jnp.where(hit, p, _N))
                    return best
                best = lax.fori_loop(0, _NV // _UNROLL, scan_chunk,
                                     jnp.full((_L,), _N, jnp.int32))
                g = _red(best, jnp.minimum)

                def write():
                    base = (g // _L) * _L
                    w = obuf[pl.ds(base, _L)]
                    obuf[pl.ds(base, _L)] = jnp.where(
                        io + base == g, rthr, w)
                lax.cond((k < r_keep) & (g < _N), write, lambda: None)
                return g
            lax.fori_loop(0, 5, patch, jnp.int32(-1))
            return 0

        lax.cond(unique, pass2_fast, pass2_exact, 0)

        pltpu.sync_copy(obuf, o_hbm.at[row])
        return ()

    lax.fori_loop(0, rows_per_worker, do_row, ())


def _sc_top5_round(x):
    rows = x.shape[0]
    rpw = rows // _NW
    mesh = plsc.VectorSubcoreMesh(
        core_axis_name="c", subcore_axis_name="s",
        num_cores=_NC, num_subcores=_NS)
    body = functools.partial(_sc_body, rows_per_worker=rpw)
    return pl.kernel(
        body,
        out_type=jax.ShapeDtypeStruct((rows, _N), jnp.float32),
        mesh=mesh,
        scratch_types=[
            pltpu.VMEM((_N,), jnp.float32),
            pltpu.VMEM((_N,), jnp.float32),
            pltpu.SemaphoreType.DMA,
        ],
    )(x)


_SC_ROWS = 32


def kernel(x):
    rows = x.shape[0]
    tc_rows = rows - _SC_ROWS
    tc_out = _tc_top5_round(x[:tc_rows])
    sc_out = _sc_top5_round(x[tc_rows:])
    return jnp.concatenate([tc_out, sc_out], axis=0)


# speedup vs baseline: 1.0822x; 1.0822x over previous
import functools

import jax
import jax.numpy as jnp
from jax import lax
from jax.experimental import pallas as pl
from jax.experimental.pallas import tpu as pltpu
from jax.experimental.pallas import tpu_sc as plsc

_ROWS = 8
_N = 32768
_CHUNK = 128
_NCHUNKS = _N // _CHUNK
_GROUPS = 4
_NEG = float("-inf")


def _insert(acc, v):
    t1, t2, t3, t4, t5 = acc
    m1 = jnp.maximum(t1, v)
    r1 = jnp.minimum(t1, v)
    m2 = jnp.maximum(t2, r1)
    r2 = jnp.minimum(t2, r1)
    m3 = jnp.maximum(t3, r2)
    r3 = jnp.minimum(t3, r2)
    m4 = jnp.maximum(t4, r3)
    r4 = jnp.minimum(t4, r3)
    m5 = jnp.maximum(t5, r4)
    return (m1, m2, m3, m4, m5)


def _fold(a, b):
    return tuple(jnp.maximum(a[i], b[4 - i]) for i in range(5))


def _chunk(ref, k):
    return ref[:, k * _CHUNK:(k + 1) * _CHUNK]


def _top5_round_body(x_ref, o_ref, scratch_ref):
    iota = jax.lax.broadcasted_iota(jnp.int32, (_ROWS, _CHUNK), 1)

    per_group = _NCHUNKS // _GROUPS
    accs = [tuple(jnp.full((_ROWS, _CHUNK), _NEG) for _ in range(5))
            for _ in range(_GROUPS)]
    for j in range(per_group):
        for g in range(_GROUPS):
            accs[g] = _insert(accs[g], _chunk(x_ref, g * per_group + j))

    folded = [_fold(accs[2 * i], accs[2 * i + 1]) for i in range(_GROUPS // 2)]

    cand = jnp.concatenate([t for f in folded for t in f], axis=1)
    ccol = jax.lax.broadcasted_iota(jnp.int32, cand.shape, 1)
    vals = []
    for _ in range(5):
        m = jnp.max(cand, axis=1, keepdims=True)
        vals.append(m)
        first_col = jnp.min(
            jnp.where(cand == m, ccol, cand.shape[1]), axis=1, keepdims=True)
        cand = jnp.where(ccol == first_col, _NEG, cand)
    v4, v5 = vals[3], vals[4]

    distinct = jnp.all(v4 > v5)

    eq_cnt = jnp.zeros((_ROWS, _CHUNK), jnp.float32)
    for g in range(_GROUPS):
        for t in accs[g]:
            eq_cnt = eq_cnt + jnp.where(t == v5, 1.0, 0.0)
    unique = jnp.sum(eq_cnt, axis=1, keepdims=True) == 1.0
    rank5_hit = jnp.zeros((_ROWS, _CHUNK), jnp.bool_)
    for g in range(_GROUPS):
        rank5_hit = rank5_hit | (accs[g][4] == v5)
    no_full_group = jnp.logical_not(jnp.any(rank5_hit, axis=1, keepdims=True))

    fast = distinct & jnp.all(unique & no_full_group)
    medium = distinct & jnp.logical_not(fast)

    @pl.when(fast)
    def _():
        for k in range(_NCHUNKS):
            v = _chunk(x_ref, k)
            o_ref[:, k * _CHUNK:(k + 1) * _CHUNK] = jnp.where(
                v >= v5, jnp.round(v), 0.0)

    @pl.when(medium)
    def _():
        parts = []
        for g in range(_GROUPS):
            m = jnp.full((_ROWS, _CHUNK), _N, jnp.int32)
            for j in range(per_group):
                k = g * per_group + j
                v = _chunk(x_ref, k)
                m = jnp.minimum(m, jnp.where(v == v5, iota + k * _CHUNK, _N))
            parts.append(m)
        first_eq = jnp.min(jnp.concatenate(parts, axis=1), axis=1,
                           keepdims=True)
        for k in range(_NCHUNKS):
            v = _chunk(x_ref, k)
            keep = (v > v5) | (iota + k * _CHUNK == first_eq)
            o_ref[:, k * _CHUNK:(k + 1) * _CHUNK] = jnp.where(
                keep, jnp.round(v), 0.0)

    @pl.when(jnp.logical_not(distinct))
    def _():
        for k in range(_NCHUNKS):
            scratch_ref[:, k * _CHUNK:(k + 1) * _CHUNK] = _chunk(x_ref, k)
        for _ in range(5):
            m = jnp.full((_ROWS, _CHUNK), _NEG)
            for k in range(_NCHUNKS):
                m = jnp.maximum(m, _chunk(scratch_ref, k))
            m = jnp.max(m, axis=1, keepdims=True)
            fc = jnp.full((_ROWS, _CHUNK), _N, jnp.int32)
            for k in range(_NCHUNKS):
                fc = jnp.minimum(fc, jnp.where(
                    _chunk(scratch_ref, k) == m, iota + k * _CHUNK, _N))
            fc = jnp.min(fc, axis=1, keepdims=True)
            for k in range(_NCHUNKS):
                w = _chunk(scratch_ref, k)
                scratch_ref[:, k * _CHUNK:(k + 1) * _CHUNK] = jnp.where(
                    iota + k * _CHUNK == fc, _NEG, w)
        for k in range(_NCHUNKS):
            v = _chunk(x_ref, k)
            sel = _chunk(scratch_ref, k) == _NEG
            o_ref[:, k * _CHUNK:(k + 1) * _CHUNK] = jnp.where(
                sel, jnp.round(v), 0.0)


def _tc_top5_round(x):
    rows, n = x.shape
    grid = (rows // _ROWS,)
    return pl.pallas_call(
        _top5_round_body,
        grid=grid,
        in_specs=[pl.BlockSpec((_ROWS, n), lambda i: (i, 0))],
        out_specs=pl.BlockSpec((_ROWS, n), lambda i: (i, 0)),
        out_shape=jax.ShapeDtypeStruct(x.shape, x.dtype),
        scratch_shapes=[pltpu.VMEM((_ROWS, _N), jnp.float32)],
    )(x)


_L = 16
_NV = _N // _L
_UNROLL = 16
_MAGIC = 12582912.0
_BIG = 4194304.0
_NC = 2
_NS = 16
_NW = _NC * _NS


def _rne(v):
    y = (v + _MAGIC) - _MAGIC
    return jnp.where(jnp.abs(v) >= _BIG, v, y)


def _red(vec, op):
    vals = [vec[j] for j in range(_L)]
    while len(vals) > 1:
        vals = [op(vals[i], vals[i + 1]) for i in range(0, len(vals), 2)]
    return vals[0]


def _sc_body(x_hbm, o_hbm, buf, obuf, sem, *, rows_per_worker):
    wid = lax.axis_index("s") * _NC + lax.axis_index("c")
    io = lax.iota(jnp.int32, _L)

    def do_row(r, _):
        row = wid * rows_per_worker + r
        pltpu.sync_copy(x_hbm.at[row], buf)

        def p1(i, acc):
            for j in range(_UNROLL):
                acc = _insert(acc, buf[pl.ds((i * _UNROLL + j) * _L, _L)])
            return acc
        acc0 = tuple(jnp.full((_L,), _NEG, jnp.float32) for _ in range(5))
        t1, t2, t3, t4, t5 = lax.fori_loop(0, _NV // _UNROLL, p1, acc0)

        cands = (t1, t2, t3, t4, t5)
        work = list(cands)
        thr = jnp.float32(0)
        for _ in range(5):
            m01 = jnp.maximum(work[0], work[1])
            m23 = jnp.maximum(work[2], work[3])
            mv = jnp.maximum(jnp.maximum(m01, m23), work[4])
            thr = _red(mv, jnp.maximum)
            pos = jnp.full((_L,), 999, jnp.int32)
            for i in range(5):
                pos = jnp.minimum(
                    pos, jnp.where(work[i] == thr, io + i * _L, 999))
            g = _red(pos, jnp.minimum)
            for i in range(5):
                work[i] = jnp.where(io + i * _L == g, _NEG, work[i])

        cgt = jnp.zeros((_L,), jnp.int32)
        ceq = jnp.zeros((_L,), jnp.int32)
        for t in cands:
            cgt = cgt + jnp.where(t > thr, 1, 0)
            ceq = ceq + jnp.where(t == thr, 1, 0)
        full5 = jnp.where(t5 == thr, 1, 0)
        count_gt = _red(cgt, jnp.add)
        eq_cand = _red(ceq, jnp.add)
        full_lanes = _red(full5, jnp.add)
        r_keep = 5 - count_gt
        unique = (eq_cand == 1) & (full_lanes == 0) & (r_keep == 1)

        rthr = _rne(thr)

        def pass2_fast(_):
            def p2(i, c):
                for j in range(_UNROLL):
                    off = (i * _UNROLL + j) * _L
                    v = buf[pl.ds(off, _L)]
                    obuf[pl.ds(off, _L)] = jnp.where(v >= thr, _rne(v), 0.0)
                return c
            lax.fori_loop(0, _NV // _UNROLL, p2, jnp.int32(0))
            return 0

        def pass2_exact(_):
            def p2(i, c):
                for j in range(_UNROLL):
                    off = (i * _UNROLL + j) * _L
                    v = buf[pl.ds(off, _L)]
                    obuf[pl.ds(off, _L)] = jnp.where(v > thr, _rne(v), 0.0)
                return c
            lax.fori_loop(0, _NV // _UNROLL, p2, jnp.int32(0))

            def patch(k, prev):
                def scan_chunk(i, best):
                    for j in range(_UNROLL):
                        off = (i * _UNROLL + j) * _L
                        v = buf[pl.ds(off, _L)]
                        p = io + off
                        hit = (v == thr) & (p > prev)
                        best = jnp.minimum(best, jnp.where(hit, p, _N))
                    return best
                best = lax.fori_loop(0, _NV // _UNROLL, scan_chunk,
                                     jnp.full((_L,), _N, jnp.int32))
                g = _red(best, jnp.minimum)

                def write():
                    base = (g // _L) * _L
                    w = obuf[pl.ds(base, _L)]
                    obuf[pl.ds(base, _L)] = jnp.where(
                        io + base == g, rthr, w)
                lax.cond((k < r_keep) & (g < _N), write, lambda: None)
                return g
            lax.fori_loop(0, 5, patch, jnp.int32(-1))
            return 0

        lax.cond(unique, pass2_fast, pass2_exact, 0)

        pltpu.sync_copy(obuf, o_hbm.at[row])
        return ()

    lax.fori_loop(0, rows_per_worker, do_row, ())


def _sc_top5_round(x):
    rows = x.shape[0]
    rpw = rows // _NW
    mesh = plsc.VectorSubcoreMesh(
        core_axis_name="c", subcore_axis_name="s",
        num_cores=_NC, num_subcores=_NS)
    body = functools.partial(_sc_body, rows_per_worker=rpw)
    return pl.kernel(
        body,
        out_type=jax.ShapeDtypeStruct((rows, _N), jnp.float32),
        mesh=mesh,
        scratch_types=[
            pltpu.VMEM((_N,), jnp.float32),
            pltpu.VMEM((_N,), jnp.float32),
            pltpu.SemaphoreType.DMA,
        ],
    )(x)


_SC_ROWS = 32


def kernel(x):
    rows = x.shape[0]
    tc_rows = rows - _SC_ROWS
    tc_out = _tc_top5_round(x[:tc_rows])
    sc_out = _sc_top5_round(x[tc_rows:])
    return jnp.concatenate([tc_out, sc_out], axis=0)
